# baseline (speedup 1.0000x reference)
import functools

import jax
import jax.numpy as jnp
from jax import lax
from jax.experimental import pallas as pl
from jax.experimental.pallas import tpu as pltpu
from jax.experimental.pallas import tpu_sc as plsc

B_TOT = 64 * 32 * 32
D = 256
NROW = 50
NC, NS = 2, 16
NW = NC * NS
BPW = B_TOT // NW
CH = 64
NCHUNK = BPW // CH

_mesh = plsc.VectorSubcoreMesh(core_axis_name="c", subcore_axis_name="s")


@functools.partial(
    pl.kernel,
    mesh=_mesh,
    out_type=jax.ShapeDtypeStruct((B_TOT, 2 * D), jnp.float32),
    scratch_types=[
        pltpu.VMEM((BPW,), jnp.int32),
        pltpu.VMEM((BPW,), jnp.int32),
        pltpu.VMEM((NROW * D,), jnp.float32),
        pltpu.VMEM((NROW * D,), jnp.float32),
        pltpu.VMEM((CH, 2 * D), jnp.float32),
        pltpu.VMEM((CH, 2 * D), jnp.float32),
        pltpu.SemaphoreType.DMA,
        pltpu.SemaphoreType.DMA,
    ],
    compiler_params=pltpu.CompilerParams(needs_layout_passes=False),
)
def _emb_lookup(i_hbm, j_hbm, col_hbm, row_hbm, out_hbm,
                i_v, j_v, col_v, row_v, b0, b1, w0, w1):
    sid = lax.axis_index("s")
    wid = sid * NC + lax.axis_index("c")
    base = wid * BPW
    pltpu.sync_copy(col_hbm, col_v)
    pltpu.sync_copy(row_hbm, row_v)
    pltpu.sync_copy(i_hbm.at[pl.ds(base, BPW)], i_v)
    pltpu.sync_copy(j_hbm.at[pl.ds(base, BPW)], j_v)

    @plsc.parallel_loop(0, BPW // 16, step=1, unroll=8)
    def _prescale(g):
        i_v[pl.ds(g * 16, 16)] = i_v[pl.ds(g * 16, 16)] * D
        j_v[pl.ds(g * 16, 16)] = j_v[pl.ds(g * 16, 16)] * D

    bufs, wsems = (b0, b1), (w0, w1)
    cols = [jnp.arange(16, dtype=jnp.int32) + 16 * k for k in range(D // 16)]

    def fill(c, buf):
        @plsc.parallel_loop(0, CH, step=1, unroll=12)
        def body(p):
            pv = jnp.broadcast_to(c * CH + p, (16,)).astype(jnp.int32)
            bi = plsc.load_gather(i_v, [pv])
            bj = plsc.load_gather(j_v, [pv])
            for k in range(D // 16):
                buf[p, pl.ds(16 * k, 16)] = plsc.load_gather(col_v, [bi + cols[k]])
                buf[p, pl.ds(D + 16 * k, 16)] = plsc.load_gather(row_v, [bj + cols[k]])

    def drain(nb):
        pltpu.make_async_copy(
            bufs[nb], out_hbm.at[pl.ds(base, CH)], wsems[nb]).wait()

    def loop_body(t, carry):
        for nb in range(2):
            c = t * 2 + nb

            @pl.when(t > 0)
            def _wait_prev():
                drain(nb)

            fill(c, bufs[nb])
            pltpu.async_copy(
                bufs[nb], out_hbm.at[pl.ds(base + c * CH, CH)], wsems[nb])
        return carry

    lax.fori_loop(0, NCHUNK // 2, loop_body, 0)
    drain(0)
    drain(1)


def kernel(i, j, row_embed, col_embed):
    out = _emb_lookup(i.reshape(-1), j.reshape(-1),
                      col_embed.reshape(-1), row_embed.reshape(-1))
    return out.reshape(64, 32, 32, 2 * D)


# speedup vs baseline: 1.3311x; 1.3311x over previous
import functools

import jax
import jax.numpy as jnp
from jax import lax
from jax.experimental import pallas as pl
from jax.experimental.pallas import tpu as pltpu
from jax.experimental.pallas import tpu_sc as plsc

B_TOT = 64 * 32 * 32
D = 256
NROW = 50
NC, NS = 2, 16
NW = NC * NS
BPW = B_TOT // NW
CH = 64
NCHUNK = BPW // CH

_mesh = plsc.VectorSubcoreMesh(core_axis_name="c", subcore_axis_name="s")


@functools.partial(
    pl.kernel,
    mesh=_mesh,
    out_type=jax.ShapeDtypeStruct((B_TOT, 2 * D), jnp.float32),
    scratch_types=[
        pltpu.VMEM((BPW,), jnp.int32),
        pltpu.VMEM((BPW,), jnp.int32),
        pltpu.VMEM((NROW * D,), jnp.float32),
        pltpu.VMEM((NROW * D,), jnp.float32),
        pltpu.VMEM((CH, 2 * D), jnp.float32),
        pltpu.VMEM((CH, 2 * D), jnp.float32),
        pltpu.SemaphoreType.DMA,
        pltpu.SemaphoreType.DMA,
    ],
    compiler_params=pltpu.CompilerParams(needs_layout_passes=False),
)
def _emb_lookup(i_hbm, j_hbm, col_hbm, row_hbm, out_hbm,
                i_v, j_v, col_v, row_v, b0, b1, w0, w1):
    sid = lax.axis_index("s")
    wid = sid * NC + lax.axis_index("c")
    base = wid * BPW
    pltpu.sync_copy(col_hbm, col_v)
    pltpu.sync_copy(row_hbm, row_v)
    pltpu.sync_copy(i_hbm.at[pl.ds(base, BPW)], i_v)
    pltpu.sync_copy(j_hbm.at[pl.ds(base, BPW)], j_v)

    @plsc.parallel_loop(0, BPW // 16, step=1, unroll=8)
    def _prescale(g):
        i_v[pl.ds(g * 16, 16)] = i_v[pl.ds(g * 16, 16)] * D
        j_v[pl.ds(g * 16, 16)] = j_v[pl.ds(g * 16, 16)] * D

    bufs, wsems = (b0, b1), (w0, w1)
    cols = [jnp.arange(16, dtype=jnp.int32) + 16 * k for k in range(D // 16)]

    def fill(c, buf):
        @plsc.parallel_loop(0, CH, step=1, unroll=8)
        def body(p):
            pv = jnp.broadcast_to(c * CH + p, (16,)).astype(jnp.int32)
            bi = plsc.load_gather(i_v, [pv])
            bj = plsc.load_gather(j_v, [pv])
            for k in range(D // 16):
                buf[p, pl.ds(16 * k, 16)] = plsc.load_gather(col_v, [bi + cols[k]])
                buf[p, pl.ds(D + 16 * k, 16)] = plsc.load_gather(row_v, [bj + cols[k]])

    def drain(nb):
        pltpu.make_async_copy(
            bufs[nb], out_hbm.at[pl.ds(base, CH)], wsems[nb]).wait()

    def loop_body(t, carry):
        for nb in range(2):
            c = t * 2 + nb

            @pl.when(t > 0)
            def _wait_prev():
                drain(nb)

            fill(c, bufs[nb])
            pltpu.async_copy(
                bufs[nb], out_hbm.at[pl.ds(base + c * CH, CH)], wsems[nb])
        return carry

    lax.fori_loop(0, NCHUNK // 2, loop_body, 0)
    drain(0)
    drain(1)


def kernel(i, j, row_embed, col_embed):
    out = _emb_lookup(i.reshape(-1), j.reshape(-1),
                      col_embed.reshape(-1), row_embed.reshape(-1))
    return out.reshape(64, 32, 32, 2 * D)
